# Initial kernel scaffold; baseline (speedup 1.0000x reference)
import functools

import jax
import jax.numpy as jnp
from jax import lax
from jax.experimental import pallas as pl
from jax.experimental.pallas import tpu as pltpu
from jax.experimental.pallas import tpu_sc as plsc

N = 10000
E = 320000
D = 128

NC = 2
NS = 16
NW = NC * NS
EPW = E // NW
CH = 125
NCHUNK = EPW // CH
RPS = N // NS


def _gather_body(x_hbm, src_hbm, dst_hbm, xs_hbm, xd_hbm,
                 idx_s, idx_d, buf_s, buf_d, sem):
    wid = lax.axis_index("s") * NC + lax.axis_index("c")
    pltpu.sync_copy(src_hbm.at[pl.ds(wid * NCHUNK, NCHUNK)], idx_s)
    pltpu.sync_copy(dst_hbm.at[pl.ds(wid * NCHUNK, NCHUNK)], idx_d)

    def step(j, carry):
        pltpu.async_copy(x_hbm.at[idx_s.at[j]], buf_s, sem).wait()
        pltpu.async_copy(x_hbm.at[idx_d.at[j]], buf_d, sem).wait()
        base = wid * EPW + j * CH
        pltpu.sync_copy(buf_s, xs_hbm.at[pl.ds(base, CH)])
        pltpu.sync_copy(buf_d, xd_hbm.at[pl.ds(base, CH)])
        return carry

    lax.fori_loop(0, NCHUNK, step, 0)


_gather = pl.kernel(
    _gather_body,
    out_type=(jax.ShapeDtypeStruct((E, D), jnp.float32),
              jax.ShapeDtypeStruct((E, D), jnp.float32)),
    mesh=plsc.VectorSubcoreMesh(core_axis_name="c", subcore_axis_name="s"),
    scratch_types=[
        pltpu.VMEM((NCHUNK, CH), jnp.int32),
        pltpu.VMEM((NCHUNK, CH), jnp.int32),
        pltpu.VMEM((CH, D), jnp.float32),
        pltpu.VMEM((CH, D), jnp.float32),
        pltpu.SemaphoreType.DMA,
    ],
)


def _mlp_body(xs_ref, xd_ref, w1_ref, b1_ref, w2r_ref, b2_ref, w4_ref,
              b4_ref, out_ref):
    xs = xs_ref[...]
    xd = xd_ref[...]
    theta = xd - xs
    h1 = jnp.dot(theta, w1_ref[...], preferred_element_type=jnp.float32)
    h1 = jnp.maximum(h1 + b1_ref[...], 0.0)
    logit = jnp.sum(h1 * w2r_ref[...], axis=1, keepdims=True) + b2_ref[0, 0]
    w = jax.nn.sigmoid(logit)
    xlin = jnp.dot(xd, w4_ref[...], preferred_element_type=jnp.float32)
    out_ref[...] = w * theta + xlin + b4_ref[...]


BE = 2000


def _mlp(xs, xd, W1, b1r, W2r, b2r, W4, b4r):
    full = lambda shape: pl.BlockSpec(shape, lambda i: (0, 0))
    return pl.pallas_call(
        _mlp_body,
        grid=(E // BE,),
        in_specs=[
            pl.BlockSpec((BE, D), lambda i: (i, 0)),
            pl.BlockSpec((BE, D), lambda i: (i, 0)),
            full((D, D)),
            full((1, D)),
            full((1, D)),
            pl.BlockSpec(memory_space=pltpu.SMEM),
            full((D, D)),
            full((1, D)),
        ],
        out_specs=pl.BlockSpec((BE, D), lambda i: (i, 0)),
        out_shape=jax.ShapeDtypeStruct((E, D), jnp.float32),
    )(xs, xd, W1, b1r, W2r, b2r, W4, b4r)


def _scatter_body(m_hbm, dst_hbm, z_hbm, out_hbm, idx_v, buf, acc, sem):
    c = lax.axis_index("c")
    s = lax.axis_index("s")
    wid = s * NC + c
    pltpu.sync_copy(z_hbm.at[pl.ds(s * RPS, RPS)], acc.at[pl.ds(s * RPS, RPS)])
    pltpu.sync_copy(dst_hbm.at[pl.ds(wid * NCHUNK, NCHUNK)], idx_v)
    plsc.subcore_barrier()

    def step(j, carry):
        base = wid * EPW + j * CH
        pltpu.sync_copy(m_hbm.at[pl.ds(base, CH)], buf)
        pltpu.sync_copy(buf, acc.at[idx_v.at[j]], add=True)
        return carry

    lax.fori_loop(0, NCHUNK, step, 0)
    plsc.subcore_barrier()
    pltpu.sync_copy(acc.at[pl.ds(s * RPS, RPS)],
                    out_hbm.at[c, pl.ds(s * RPS, RPS)])


_scatter = pl.kernel(
    _scatter_body,
    out_type=jax.ShapeDtypeStruct((NC, N, D), jnp.float32),
    mesh=plsc.VectorSubcoreMesh(core_axis_name="c", subcore_axis_name="s"),
    scratch_types=[
        pltpu.VMEM((NCHUNK, CH), jnp.int32),
        pltpu.VMEM((CH, D), jnp.float32),
        pltpu.VMEM_SHARED((N, D), jnp.float32),
        pltpu.SemaphoreType.DMA,
    ],
)


def _combine_body(p_ref, out_ref):
    out_ref[...] = p_ref[0] + p_ref[1]


BN = 1000


def _combine(partials):
    return pl.pallas_call(
        _combine_body,
        grid=(N // BN,),
        in_specs=[pl.BlockSpec((NC, BN, D), lambda i: (0, i, 0))],
        out_specs=pl.BlockSpec((BN, D), lambda i: (i, 0)),
        out_shape=jax.ShapeDtypeStruct((N, D), jnp.float32),
    )(partials)


def kernel(x, edge_index, W1, b1, W2, b2, W4, b4):
    src2 = edge_index[0].reshape(NW * NCHUNK, CH)
    dst2 = edge_index[1].reshape(NW * NCHUNK, CH)
    xs, xd = _gather(x, src2, dst2)
    msg = _mlp(xs, xd, W1, b1.reshape(1, D), W2.reshape(1, D),
               b2.reshape(1, 1), W4, b4.reshape(1, D))
    partials = _scatter(msg, dst2, jnp.zeros((N, D), jnp.float32))
    return _combine(partials)


# speedup vs baseline: 4.0543x; 4.0543x over previous
import jax
import jax.numpy as jnp
from jax import lax
from jax.experimental import pallas as pl
from jax.experimental.pallas import tpu as pltpu
from jax.experimental.pallas import tpu_sc as plsc

N = 10000
E = 320000
D = 128

NC = 2
NS = 16
NW = NC * NS
EPW = E // NW
CH = 80
NCHUNK = EPW // CH
NPAD = 10240
RPS = NPAD // NS


def _gather_body(x_hbm, src_hbm, dst_hbm, xs_hbm, xd_hbm,
                 idx_s, idx_d, buf_s, buf_d, sem):
    wid = lax.axis_index("s") * NC + lax.axis_index("c")
    pltpu.sync_copy(src_hbm.at[wid], idx_s)
    pltpu.sync_copy(dst_hbm.at[wid], idx_d)

    def step(j, carry):
        pltpu.async_copy(x_hbm.at[idx_s.at[j]], buf_s, sem).wait()
        pltpu.async_copy(x_hbm.at[idx_d.at[j]], buf_d, sem).wait()
        base = wid * EPW + j * CH
        pltpu.sync_copy(buf_s, xs_hbm.at[pl.ds(base, CH)])
        pltpu.sync_copy(buf_d, xd_hbm.at[pl.ds(base, CH)])
        return carry

    lax.fori_loop(0, NCHUNK, step, 0)


_gather = pl.kernel(
    _gather_body,
    out_type=(jax.ShapeDtypeStruct((E, D), jnp.float32),
              jax.ShapeDtypeStruct((E, D), jnp.float32)),
    mesh=plsc.VectorSubcoreMesh(core_axis_name="c", subcore_axis_name="s"),
    scratch_types=[
        pltpu.VMEM((NCHUNK, CH), jnp.int32),
        pltpu.VMEM((NCHUNK, CH), jnp.int32),
        pltpu.VMEM((CH, D), jnp.float32),
        pltpu.VMEM((CH, D), jnp.float32),
        pltpu.SemaphoreType.DMA,
    ],
)


def _mlp_body(xs_ref, xd_ref, w1_ref, b1_ref, w2r_ref, b2_ref, w4_ref,
              b4_ref, out_ref):
    xs = xs_ref[...]
    xd = xd_ref[...]
    theta = xd - xs
    h1 = jnp.dot(theta, w1_ref[...], preferred_element_type=jnp.float32)
    h1 = jnp.maximum(h1 + b1_ref[...], 0.0)
    logit = jnp.sum(h1 * w2r_ref[...], axis=1, keepdims=True) + b2_ref[0, 0]
    w = jax.nn.sigmoid(logit)
    xlin = jnp.dot(xd, w4_ref[...], preferred_element_type=jnp.float32)
    out_ref[...] = w * theta + xlin + b4_ref[...]


BE = 2000


def _mlp(xs, xd, W1, b1r, W2r, b2r, W4, b4r):
    full = lambda shape: pl.BlockSpec(shape, lambda i: (0, 0))
    return pl.pallas_call(
        _mlp_body,
        grid=(E // BE,),
        in_specs=[
            pl.BlockSpec((BE, D), lambda i: (i, 0)),
            pl.BlockSpec((BE, D), lambda i: (i, 0)),
            full((D, D)),
            full((1, D)),
            full((1, D)),
            pl.BlockSpec(memory_space=pltpu.SMEM),
            full((D, D)),
            full((1, D)),
        ],
        out_specs=pl.BlockSpec((BE, D), lambda i: (i, 0)),
        out_shape=jax.ShapeDtypeStruct((E, D), jnp.float32),
    )(xs, xd, W1, b1r, W2r, b2r, W4, b4r)


def _scatter_body(m_hbm, dst_hbm, z_hbm, out_hbm, idx_v, buf, acc, sem):
    c = lax.axis_index("c")
    s = lax.axis_index("s")
    wid = s * NC + c
    pltpu.sync_copy(z_hbm.at[pl.ds(s * RPS, RPS)], acc.at[pl.ds(s * RPS, RPS)])
    pltpu.sync_copy(dst_hbm.at[wid], idx_v)
    plsc.subcore_barrier()

    def step(j, carry):
        base = wid * EPW + j * CH
        pltpu.sync_copy(m_hbm.at[pl.ds(base, CH)], buf)
        pltpu.sync_copy(buf, acc.at[idx_v.at[j]], add=True)
        return carry

    lax.fori_loop(0, NCHUNK, step, 0)
    plsc.subcore_barrier()
    pltpu.sync_copy(acc.at[pl.ds(s * RPS, RPS)],
                    out_hbm.at[c, pl.ds(s * RPS, RPS)])


_scatter = pl.kernel(
    _scatter_body,
    out_type=jax.ShapeDtypeStruct((NC, NPAD, D), jnp.float32),
    mesh=plsc.VectorSubcoreMesh(core_axis_name="c", subcore_axis_name="s"),
    scratch_types=[
        pltpu.VMEM((NCHUNK, CH), jnp.int32),
        pltpu.VMEM((CH, D), jnp.float32),
        pltpu.VMEM_SHARED((NPAD, D), jnp.float32),
        pltpu.SemaphoreType.DMA,
    ],
)


def _combine_body(p_ref, out_ref):
    out_ref[...] = p_ref[0] + p_ref[1]


BN = 1000


def _combine(partials):
    return pl.pallas_call(
        _combine_body,
        grid=(N // BN,),
        in_specs=[pl.BlockSpec((NC, BN, D), lambda i: (0, i, 0))],
        out_specs=pl.BlockSpec((BN, D), lambda i: (i, 0)),
        out_shape=jax.ShapeDtypeStruct((N, D), jnp.float32),
    )(partials)


def kernel(x, edge_index, W1, b1, W2, b2, W4, b4):
    src3 = edge_index[0].reshape(NW, NCHUNK, CH)
    dst3 = edge_index[1].reshape(NW, NCHUNK, CH)
    xs, xd = _gather(x, src3, dst3)
    msg = _mlp(xs, xd, W1, b1.reshape(1, D), W2.reshape(1, D),
               b2.reshape(1, 1), W4, b4.reshape(1, D))
    partials = _scatter(msg, dst3, jnp.zeros((NPAD, D), jnp.float32))
    return _combine(partials)
